# baseline (speedup 1.0000x reference)
import functools

import jax
import jax.numpy as jnp
from jax import lax
from jax.experimental import pallas as pl
from jax.experimental.pallas import tpu as pltpu
from jax.experimental.pallas import tpu_sc as plsc

L = 16


def _ceil_div(a, b):
    return (a + b - 1) // b


@functools.lru_cache(maxsize=None)
def _make_sc_call(N, D, B, CH=128):
    assert D % L == 0
    info = plsc.get_sparse_core_info()
    NC, NS = info.num_cores, info.num_subcores
    NW = NC * NS
    RPT = _ceil_div(_ceil_div(N, NW), 8) * 8
    NCHUNK = _ceil_div(RPT, CH)
    SL = (B * D) // NS
    mesh = plsc.VectorSubcoreMesh(core_axis_name="c", subcore_axis_name="s")

    @functools.partial(
        pl.kernel,
        mesh=mesh,
        out_type=jax.ShapeDtypeStruct((NC, B, D), jnp.float32),
        compiler_params=pltpu.CompilerParams(needs_layout_passes=False),
        scratch_types=dict(
            buf0=pltpu.VMEM((CH * D,), jnp.float32),
            buf1=pltpu.VMEM((CH * D,), jnp.float32),
            bbuf=pltpu.VMEM((RPT,), jnp.int32),
            g_v=pltpu.VMEM((B, D), jnp.float32),
            acc_v=pltpu.VMEM((B, D), jnp.float32),
            shared=pltpu.VMEM_SHARED((B, D), jnp.float32),
            sem0=pltpu.SemaphoreType.DMA,
            sem1=pltpu.SemaphoreType.DMA,
            semb0=pltpu.SemaphoreType.DMA,
            semb1=pltpu.SemaphoreType.DMA,
        ),
    )
    def sc_kernel(
        nemb_hbm, g_hbm, nbatch_hbm, out_hbm,
        buf0, buf1, bbuf, g_v, acc_v, shared, sem0, sem1, semb0, semb1,
    ):
        cid = lax.axis_index("c")
        sid = lax.axis_index("s")
        wid = cid * NS + sid
        lo = wid * RPT
        hi = jnp.minimum(lo + RPT, N)

        cols = [lax.iota(jnp.int32, L) + j * L for j in range(D // L)]
        zeros = jnp.zeros((L,), jnp.float32)

        def zero_body(r, _):
            rsp = jnp.full((L,), r, jnp.int32)
            for j in range(D // L):
                plsc.store_scatter(acc_v, [rsp, cols[j]], zeros)
            return 0

        cs_all = jnp.minimum(lo, N - RPT)
        pltpu.async_copy(nbatch_hbm.at[pl.ds(cs_all, RPT)], bbuf, semb0)

        def start_chunk(c, bufx, semx):
            start_r = lo + c * CH
            cs = jnp.minimum(start_r, N - CH)
            pltpu.async_copy(nemb_hbm.at[pl.ds(cs * D, CH * D)], bufx, semx)

        def wait_chunk(bufx, semx):
            pltpu.make_async_copy(nemb_hbm.at[pl.ds(0, CH * D)], bufx, semx).wait()

        start_chunk(0, buf0, sem0)

        pltpu.sync_copy(g_hbm, g_v)
        lax.fori_loop(0, B, zero_body, 0)

        @pl.when(sid == 0)
        def _():
            pltpu.sync_copy(acc_v, shared)

        plsc.subcore_barrier()
        pltpu.make_async_copy(nbatch_hbm.at[pl.ds(0, RPT)], bbuf, semb0).wait()

        def compute_chunk(c, bufx):
            start_r = lo + c * CH
            cs = jnp.minimum(start_r, N - CH)
            lo_i = start_r - cs
            hi_i = jnp.minimum(jnp.minimum(start_r + CH, hi) - cs, CH)
            boff = cs - cs_all

            def row_body(i):
                bsp = plsc.load_gather(bbuf, [jnp.full((L,), boff + i, jnp.int32)])
                xs = []
                part = None
                for j in range(D // L):
                    x = bufx[pl.ds(i * D + j * L, L)]
                    g = plsc.load_gather(g_v, [bsp, cols[j]])
                    xs.append(x)
                    p = x * g
                    part = p if part is None else part + p
                tot = jnp.sum(part)
                cvec = 1.0 / (1.0 + jnp.exp(jnp.full((L,), -tot)))
                for j in range(D // L):
                    plsc.addupdate_scatter(acc_v, [bsp, cols[j]], cvec * xs[j])

            def fast16(i0, bsp0):
                gs = [plsc.load_gather(g_v, [bsp0, cols[j]]) for j in range(D // L)]
                waccs = [zeros] * (D // L)
                for r in range(L):
                    i = i0 + r
                    xs = [bufx[pl.ds(i * D + j * L, L)] for j in range(D // L)]
                    ps = [xs[j] * gs[j] for j in range(D // L)]
                    q = [ps[0] + ps[1], ps[2] + ps[3], ps[4] + ps[5], ps[6] + ps[7]]
                    part = ((q[0] + q[1]) + q[2]) + q[3]
                    tot = jnp.sum(part)
                    cvec = 1.0 / (1.0 + jnp.exp(jnp.full((L,), -tot)))
                    for j in range(D // L):
                        waccs[j] = waccs[j] + cvec * xs[j]
                for j in range(D // L):
                    plsc.addupdate_scatter(acc_v, [bsp0, cols[j]], waccs[j])

            def group_body(gidx):
                i0 = gidx * L
                bvec = bbuf[pl.ds(boff + i0, L)]
                bsp0 = plsc.load_gather(bbuf, [jnp.full((L,), boff + i0, jnp.int32)])
                uniform = jnp.all(bvec == bsp0)

                def fast(_):
                    fast16(i0, bsp0)
                    return 0

                def slow(_):
                    def rb(i, _):
                        row_body(i)
                        return 0

                    lax.fori_loop(i0, i0 + L, rb, 0)
                    return 0

                lax.cond(uniform, fast, slow, 0)

            def full_path(_):
                bF = plsc.load_gather(bbuf, [jnp.full((L,), boff, jnp.int32)])
                bLst = plsc.load_gather(
                    bbuf, [jnp.full((L,), boff + CH - 1, jnp.int32)]
                )

                def chunk_uniform(_):
                    def gb(gidx):
                        fast16(gidx * L, bF)

                    plsc.parallel_loop(0, CH // L, 1)(gb)
                    return 0

                def chunk_mixed(_):
                    plsc.parallel_loop(0, CH // L, 1)(group_body)
                    return 0

                lax.cond(jnp.all(bF == bLst), chunk_uniform, chunk_mixed, 0)
                return 0

            def partial_path(_):
                def rb(i, _):
                    row_body(i)
                    return 0

                lax.fori_loop(lo_i, hi_i, rb, 0)
                return 0

            lax.cond((lo_i == 0) & (hi_i == CH), full_path, partial_path, 0)

        def pair_body(p, _):
            c0 = 2 * p
            wait_chunk(buf0, sem0)
            start_chunk(c0 + 1, buf1, sem1)
            compute_chunk(c0, buf0)
            wait_chunk(buf1, sem1)
            start_chunk(c0 + 2, buf0, sem0)
            compute_chunk(c0 + 1, buf1)
            return 0

        lax.fori_loop(0, _ceil_div(NCHUNK, 2), pair_body, 0)
        wait_chunk(buf0, sem0)

        adds = []
        for k in range(B // L):
            idx = lax.iota(jnp.int32, L) + k * L
            adds.append(
                pltpu.async_copy(acc_v.at[pl.ds(k * L, L), :], shared.at[idx], sem0, add=True)
            )
        for cp in adds:
            cp.wait()

        plsc.subcore_barrier()

        @pl.when(sid == 0)
        def _():
            pltpu.sync_copy(shared, out_hbm.at[cid])

    return sc_kernel


def _tc_add(partials):
    NC, B, D = partials.shape

    def body(x_ref, o_ref):
        acc = x_ref[0]
        for c in range(1, NC):
            acc = acc + x_ref[c]
        o_ref[...] = acc

    return pl.pallas_call(
        body, out_shape=jax.ShapeDtypeStruct((B, D), jnp.float32)
    )(partials)


def kernel(n_embedding, g_embedding, n_batch, size):
    N, D = n_embedding.shape
    B = g_embedding.shape[0]
    nb = n_batch.astype(jnp.int32)
    sc_call = _make_sc_call(N, D, B)
    partials = sc_call(n_embedding.reshape(N * D), g_embedding, nb)
    return _tc_add(partials)


# speedup vs baseline: 1.5631x; 1.5631x over previous
import functools

import jax
import jax.numpy as jnp
from jax import lax
from jax.experimental import pallas as pl
from jax.experimental.pallas import tpu as pltpu
from jax.experimental.pallas import tpu_sc as plsc

L = 16


def _ceil_div(a, b):
    return (a + b - 1) // b


@functools.lru_cache(maxsize=None)
def _make_sc_call(N, D, B, CH=128):
    assert D % L == 0
    info = plsc.get_sparse_core_info()
    NC, NS = info.num_cores, info.num_subcores
    NW = NC * NS
    RPT = _ceil_div(_ceil_div(N, NW), 8) * 8
    NCHUNK = _ceil_div(RPT, CH)
    SL = (B * D) // NS
    mesh = plsc.VectorSubcoreMesh(core_axis_name="c", subcore_axis_name="s")

    @functools.partial(
        pl.kernel,
        mesh=mesh,
        out_type=jax.ShapeDtypeStruct((NC, B, D), jnp.float32),
        compiler_params=pltpu.CompilerParams(needs_layout_passes=False),
        scratch_types=dict(
            buf0=pltpu.VMEM((CH * D,), jnp.float32),
            buf1=pltpu.VMEM((CH * D,), jnp.float32),
            bbuf0=pltpu.VMEM((CH,), jnp.int32),
            bbuf1=pltpu.VMEM((CH,), jnp.int32),
            g_v=pltpu.VMEM((B, D), jnp.float32),
            acc_v=pltpu.VMEM((B, D), jnp.float32),
            shared=pltpu.VMEM_SHARED((B, D), jnp.float32),
            sem0=pltpu.SemaphoreType.DMA,
            sem1=pltpu.SemaphoreType.DMA,
            semb0=pltpu.SemaphoreType.DMA,
            semb1=pltpu.SemaphoreType.DMA,
        ),
    )
    def sc_kernel(
        nemb_hbm, g_hbm, nbatch_hbm, out_hbm,
        buf0, buf1, bbuf0, bbuf1, g_v, acc_v, shared, sem0, sem1, semb0, semb1,
    ):
        cid = lax.axis_index("c")
        sid = lax.axis_index("s")
        wid = cid * NS + sid
        lo = wid * RPT
        hi = jnp.minimum(lo + RPT, N)

        cols = [lax.iota(jnp.int32, L) + j * L for j in range(D // L)]
        zeros = jnp.zeros((L,), jnp.float32)

        def zero_body(r, _):
            rsp = jnp.full((L,), r, jnp.int32)
            for j in range(D // L):
                plsc.store_scatter(acc_v, [rsp, cols[j]], zeros)
            return 0

        def start_chunk(c, bufx, bbufx, semx, sembx):
            start_r = lo + c * CH
            cs = jnp.minimum(start_r, N - CH)
            pltpu.async_copy(nemb_hbm.at[pl.ds(cs * D, CH * D)], bufx, semx)
            pltpu.async_copy(nbatch_hbm.at[pl.ds(cs, CH)], bbufx, sembx)

        def wait_chunk(bufx, bbufx, semx, sembx):
            pltpu.make_async_copy(nemb_hbm.at[pl.ds(0, CH * D)], bufx, semx).wait()
            pltpu.make_async_copy(nbatch_hbm.at[pl.ds(0, CH)], bbufx, sembx).wait()

        start_chunk(0, buf0, bbuf0, sem0, semb0)

        pltpu.sync_copy(g_hbm, g_v)
        lax.fori_loop(0, B, zero_body, 0)

        @pl.when(sid == 0)
        def _():
            pltpu.sync_copy(acc_v, shared)

        plsc.subcore_barrier()

        def compute_chunk(c, bufx, bbufx):
            start_r = lo + c * CH
            cs = jnp.minimum(start_r, N - CH)
            lo_i = start_r - cs
            hi_i = jnp.minimum(jnp.minimum(start_r + CH, hi) - cs, CH)

            def row_body(i):
                bsp = plsc.load_gather(bbufx, [jnp.full((L,), i, jnp.int32)])
                xs = []
                part = None
                for j in range(D // L):
                    x = bufx[pl.ds(i * D + j * L, L)]
                    g = plsc.load_gather(g_v, [bsp, cols[j]])
                    xs.append(x)
                    p = x * g
                    part = p if part is None else part + p
                tot = jnp.sum(part)
                cvec = 1.0 / (1.0 + jnp.exp(jnp.full((L,), -tot)))
                for j in range(D // L):
                    plsc.addupdate_scatter(acc_v, [bsp, cols[j]], cvec * xs[j])

            def group_body(gidx):
                i0 = gidx * L
                bvec = bbufx[pl.ds(i0, L)]
                bsp0 = plsc.load_gather(bbufx, [jnp.full((L,), i0, jnp.int32)])
                uniform = jnp.all(bvec == bsp0)

                def fast(_):
                    gs = [plsc.load_gather(g_v, [bsp0, cols[j]]) for j in range(D // L)]
                    waccs = [zeros] * (D // L)
                    for r in range(L):
                        i = i0 + r
                        xs = [bufx[pl.ds(i * D + j * L, L)] for j in range(D // L)]
                        ps = [xs[j] * gs[j] for j in range(D // L)]
                        q = [ps[0] + ps[1], ps[2] + ps[3], ps[4] + ps[5], ps[6] + ps[7]]
                        part = ((q[0] + q[1]) + q[2]) + q[3]
                        tot = jnp.sum(part)
                        cvec = 1.0 / (1.0 + jnp.exp(jnp.full((L,), -tot)))
                        for j in range(D // L):
                            waccs[j] = waccs[j] + cvec * xs[j]
                    for j in range(D // L):
                        plsc.addupdate_scatter(acc_v, [bsp0, cols[j]], waccs[j])
                    return 0

                def slow(_):
                    def rb(i, _):
                        row_body(i)
                        return 0

                    lax.fori_loop(i0, i0 + L, rb, 0)
                    return 0

                lax.cond(uniform, fast, slow, 0)

            def full_path(_):
                plsc.parallel_loop(0, CH // L, 1)(group_body)
                return 0

            def partial_path(_):
                def rb(i, _):
                    row_body(i)
                    return 0

                lax.fori_loop(lo_i, hi_i, rb, 0)
                return 0

            lax.cond((lo_i == 0) & (hi_i == CH), full_path, partial_path, 0)

        def pair_body(p, _):
            c0 = 2 * p
            wait_chunk(buf0, bbuf0, sem0, semb0)
            start_chunk(c0 + 1, buf1, bbuf1, sem1, semb1)
            compute_chunk(c0, buf0, bbuf0)
            wait_chunk(buf1, bbuf1, sem1, semb1)
            start_chunk(c0 + 2, buf0, bbuf0, sem0, semb0)
            compute_chunk(c0 + 1, buf1, bbuf1)
            return 0

        lax.fori_loop(0, _ceil_div(NCHUNK, 2), pair_body, 0)
        wait_chunk(buf0, bbuf0, sem0, semb0)

        adds = []
        for k in range(B // L):
            idx = lax.iota(jnp.int32, L) + k * L
            adds.append(
                pltpu.async_copy(acc_v.at[pl.ds(k * L, L), :], shared.at[idx], sem0, add=True)
            )
        for cp in adds:
            cp.wait()

        plsc.subcore_barrier()

        @pl.when(sid == 0)
        def _():
            pltpu.sync_copy(shared, out_hbm.at[cid])

    return sc_kernel


def _tc_add(partials):
    NC, B, D = partials.shape

    def body(x_ref, o_ref):
        acc = x_ref[0]
        for c in range(1, NC):
            acc = acc + x_ref[c]
        o_ref[...] = acc

    return pl.pallas_call(
        body, out_shape=jax.ShapeDtypeStruct((B, D), jnp.float32)
    )(partials)


def kernel(n_embedding, g_embedding, n_batch, size):
    N, D = n_embedding.shape
    B = g_embedding.shape[0]
    nb = n_batch.astype(jnp.int32)
    sc_call = _make_sc_call(N, D, B)
    partials = sc_call(n_embedding.reshape(N * D), g_embedding, nb)
    return _tc_add(partials)
